# baseline (speedup 1.0000x reference)
import functools

import jax
import jax.numpy as jnp
from jax import lax
from jax.experimental import pallas as pl
from jax.experimental.pallas import tpu as pltpu
from jax.experimental.pallas import tpu_sc as plsc

NUM_ITEMS = 1000000
B = 4096
S = 50
S_PAD = 56
E = 64
NW = 32

ITEMS_PER_TILE = (B * S) // NW
ITEM_CHUNK = 320
N_ITEM_CHUNKS = ITEMS_PER_TILE // ITEM_CHUNK
B_PER_TILE = B // NW
B_GROUP = 8
N_B_GROUPS = B_PER_TILE // B_GROUP
CTX_CHUNK = B_GROUP * S_PAD

_MESH = plsc.VectorSubcoreMesh(core_axis_name="c", subcore_axis_name="s")
_SC_PARAMS = pltpu.CompilerParams(use_tc_tiling_on_sc=False)


def _sc_item_gather(itab, iidx_flat):
    @functools.partial(
        pl.kernel,
        mesh=_MESH,
        out_type=jax.ShapeDtypeStruct((B * S, 2 * E), jnp.float32),
        scratch_types=[
            pltpu.VMEM((ITEM_CHUNK,), jnp.int32),
            pltpu.VMEM((ITEM_CHUNK,), jnp.int32),
            pltpu.VMEM((ITEM_CHUNK, E), jnp.float32),
            pltpu.VMEM((ITEM_CHUNK, E), jnp.float32),
            pltpu.SemaphoreType.DMA,
            pltpu.SemaphoreType.DMA,
            pltpu.SemaphoreType.DMA,
            pltpu.SemaphoreType.DMA,
        ],
        compiler_params=_SC_PARAMS,
    )
    def k(itab_hbm, iidx_hbm, item_out,
          iidx0, iidx1, ibuf0, ibuf1, sem_g0, sem_g1, sem_w0, sem_w1):
        wid = lax.axis_index("s") * 2 + lax.axis_index("c")
        iidx = (iidx0, iidx1)
        ibuf = (ibuf0, ibuf1)
        sem_g = (sem_g0, sem_g1)
        sem_w = (sem_w0, sem_w1)

        def istart(kk):
            bsel = kk % 2
            base = pl.multiple_of(wid * ITEMS_PER_TILE + kk * ITEM_CHUNK,
                                  ITEM_CHUNK)
            pltpu.sync_copy(iidx_hbm.at[pl.ds(base, ITEM_CHUNK)], iidx[bsel])
            return pltpu.async_copy(itab_hbm.at[iidx[bsel]], ibuf[bsel],
                                    sem_g[bsel])

        def iwrite(kk):
            bsel = kk % 2
            base = pl.multiple_of(wid * ITEMS_PER_TILE + kk * ITEM_CHUNK,
                                  ITEM_CHUNK)
            return pltpu.async_copy(
                ibuf[bsel],
                item_out.at[pl.ds(base, ITEM_CHUNK), pl.ds(0, E)],
                sem_w[bsel])

        gathers = [istart(0)]
        writes = []
        for kk in range(1, N_ITEM_CHUNKS):
            if kk >= 2:
                writes[kk - 2].wait()
            gathers.append(istart(kk))
            gathers[kk - 1].wait()
            writes.append(iwrite(kk - 1))
        gathers[N_ITEM_CHUNKS - 1].wait()
        writes.append(iwrite(N_ITEM_CHUNKS - 1))
        writes[N_ITEM_CHUNKS - 2].wait()
        writes[N_ITEM_CHUNKS - 1].wait()

    return k(itab, iidx_flat)


def _sc_ctx_pool(ctab, cidx_flat):
    @functools.partial(
        pl.kernel,
        mesh=_MESH,
        out_type=jax.ShapeDtypeStruct((B, E), jnp.float32),
        scratch_types=[
            pltpu.VMEM((CTX_CHUNK,), jnp.int32),
            pltpu.VMEM((CTX_CHUNK,), jnp.int32),
            pltpu.VMEM((CTX_CHUNK, E), jnp.float32),
            pltpu.VMEM((CTX_CHUNK, E), jnp.float32),
            pltpu.VMEM((B_GROUP, E), jnp.float32),
            pltpu.VMEM((B_GROUP, E), jnp.float32),
            pltpu.SemaphoreType.DMA,
            pltpu.SemaphoreType.DMA,
            pltpu.SemaphoreType.DMA,
            pltpu.SemaphoreType.DMA,
        ],
        compiler_params=_SC_PARAMS,
    )
    def k(ctab_hbm, cidx_hbm, pooled_out,
          cidx0, cidx1, cbuf0, cbuf1, pool0, pool1,
          sem_g0, sem_g1, sem_p0, sem_p1):
        wid = lax.axis_index("s") * 2 + lax.axis_index("c")
        cidx = (cidx0, cidx1)
        cbuf = (cbuf0, cbuf1)
        pool = (pool0, pool1)
        sem_g = (sem_g0, sem_g1)
        sem_p = (sem_p0, sem_p1)

        def cstart(g):
            bsel = g % 2
            cbase = pl.multiple_of(
                wid * (B_PER_TILE * S_PAD) + g * CTX_CHUNK, CTX_CHUNK)
            pltpu.sync_copy(cidx_hbm.at[pl.ds(cbase, CTX_CHUNK)], cidx[bsel])
            return pltpu.async_copy(ctab_hbm.at[cidx[bsel]], cbuf[bsel],
                                    sem_g[bsel])

        cgathers = [cstart(0)]
        pwrites = []
        for g in range(N_B_GROUPS):
            psel = g % 2
            bsel = g % 2
            cgathers[g].wait()
            if g + 1 < N_B_GROUPS:
                cgathers.append(cstart(g + 1))
            if g >= 2:
                pwrites[g - 2].wait()
            zero = jnp.zeros((16,), jnp.float32)
            for bi in range(B_GROUP):
                def body(s, acc, _bi=bi, _bsel=bsel):
                    j = _bi * S_PAD + s
                    return tuple(
                        acc[c] + cbuf[_bsel][j, pl.ds(c * 16, 16)]
                        for c in range(4))

                acc = lax.fori_loop(0, S, body, (zero, zero, zero, zero))
                for c in range(4):
                    pool[psel][bi, pl.ds(c * 16, 16)] = acc[c]
            obase = pl.multiple_of(wid * B_PER_TILE + g * B_GROUP, B_GROUP)
            pwrites.append(pltpu.async_copy(
                pool[psel], pooled_out.at[pl.ds(obase, B_GROUP)],
                sem_p[psel]))
        pwrites[N_B_GROUPS - 2].wait()
        pwrites[N_B_GROUPS - 1].wait()

    return k(ctab, cidx_flat)


def _tc_score(item2, cidx, pooled, row0, W1, b1, W2, b2):
    BB = 512

    def body(item_ref, cidx_ref, pool_ref, row0_ref, W1_ref, b1_ref,
             W2_ref, b2_ref, out_ref):
        nz = jnp.sum((cidx_ref[...] == 0).astype(jnp.float32), axis=1,
                     keepdims=True)
        pooled_c = pool_ref[...] - nz * row0_ref[...]
        up = jnp.tanh(
            jnp.dot(pooled_c, W1_ref[...],
                    preferred_element_type=jnp.float32) + b1_ref[...])
        ctx = (jnp.dot(up, W2_ref[...], preferred_element_type=jnp.float32)
               + b2_ref[...])
        item3 = item_ref[...][:, :E].reshape(BB, S, E)
        diff = item3 - ctx[:, None, :]
        d2 = jnp.sum(diff * diff, axis=-1)
        out_ref[...] = 1.0 - jnp.tanh(jnp.sqrt(d2))

    return pl.pallas_call(
        body,
        grid=(B // BB,),
        in_specs=[
            pl.BlockSpec((BB * S, 2 * E), lambda i: (i, 0)),
            pl.BlockSpec((BB, S), lambda i: (i, 0)),
            pl.BlockSpec((BB, E), lambda i: (i, 0)),
            pl.BlockSpec((1, E), lambda i: (0, 0)),
            pl.BlockSpec((E, 2 * E), lambda i: (0, 0)),
            pl.BlockSpec((1, 2 * E), lambda i: (0, 0)),
            pl.BlockSpec((2 * E, E), lambda i: (0, 0)),
            pl.BlockSpec((1, E), lambda i: (0, 0)),
        ],
        out_specs=pl.BlockSpec((BB, S), lambda i: (i, 0)),
        out_shape=jax.ShapeDtypeStruct((B, S), jnp.float32),
    )(item2, cidx, pooled, row0, W1, b1, W2, b2)


def kernel(item_indices, context_indices, item_table, ctx_table, W1, b1, W2, b2):
    ii = item_indices.astype(jnp.int32)
    ci = context_indices.astype(jnp.int32)
    spread = (jnp.arange(B * (S_PAD - S), dtype=jnp.int32) * 7919) % NUM_ITEMS
    cidx_pad = jnp.concatenate([ci, spread.reshape(B, S_PAD - S)], axis=1)
    item_embeds = _sc_item_gather(item_table, ii.reshape(-1))
    pooled = _sc_ctx_pool(ctx_table, cidx_pad.reshape(-1))
    row0 = lax.slice(ctx_table, (0, 0), (1, E))
    return _tc_score(item_embeds, ci, pooled, row0,
                     W1, b1.reshape(1, -1), W2, b2.reshape(1, -1))


# speedup vs baseline: 1.0013x; 1.0013x over previous
import functools

import jax
import jax.numpy as jnp
from jax import lax
from jax.experimental import pallas as pl
from jax.experimental.pallas import tpu as pltpu
from jax.experimental.pallas import tpu_sc as plsc

NUM_ITEMS = 1000000
B = 4096
S = 50
S_PAD = 56
E = 64
NW = 32

ITEMS_PER_TILE = (B * S) // NW
ITEM_CHUNK = 320
N_ITEM_CHUNKS = ITEMS_PER_TILE // ITEM_CHUNK
B_PER_TILE = B // NW
B_GROUP = 8
N_B_GROUPS = B_PER_TILE // B_GROUP
CTX_CHUNK = B_GROUP * S_PAD

_MESH = plsc.VectorSubcoreMesh(core_axis_name="c", subcore_axis_name="s")
_SC_PARAMS = pltpu.CompilerParams(use_tc_tiling_on_sc=False)


def _sc_item_gather(itab, iidx_flat):
    @functools.partial(
        pl.kernel,
        mesh=_MESH,
        out_type=jax.ShapeDtypeStruct((B * S, 2 * E), jnp.float32),
        scratch_types=[
            pltpu.VMEM((ITEM_CHUNK,), jnp.int32),
            pltpu.VMEM((ITEM_CHUNK,), jnp.int32),
            pltpu.VMEM((ITEM_CHUNK, E), jnp.float32),
            pltpu.VMEM((ITEM_CHUNK, E), jnp.float32),
            pltpu.SemaphoreType.DMA,
            pltpu.SemaphoreType.DMA,
            pltpu.SemaphoreType.DMA,
            pltpu.SemaphoreType.DMA,
        ],
        compiler_params=_SC_PARAMS,
    )
    def k(itab_hbm, iidx_hbm, item_out,
          iidx0, iidx1, ibuf0, ibuf1, sem_g0, sem_g1, sem_w0, sem_w1):
        wid = lax.axis_index("s") * 2 + lax.axis_index("c")
        iidx = (iidx0, iidx1)
        ibuf = (ibuf0, ibuf1)
        sem_g = (sem_g0, sem_g1)
        sem_w = (sem_w0, sem_w1)

        def istart(kk):
            bsel = kk % 2
            base = pl.multiple_of(wid * ITEMS_PER_TILE + kk * ITEM_CHUNK,
                                  ITEM_CHUNK)
            pltpu.sync_copy(iidx_hbm.at[pl.ds(base, ITEM_CHUNK)], iidx[bsel])
            return pltpu.async_copy(itab_hbm.at[iidx[bsel]], ibuf[bsel],
                                    sem_g[bsel])

        def iwrite(kk):
            bsel = kk % 2
            base = pl.multiple_of(wid * ITEMS_PER_TILE + kk * ITEM_CHUNK,
                                  ITEM_CHUNK)
            return pltpu.async_copy(
                ibuf[bsel],
                item_out.at[pl.ds(base, ITEM_CHUNK), pl.ds(0, E)],
                sem_w[bsel])

        gathers = [istart(0)]
        writes = []
        for kk in range(1, N_ITEM_CHUNKS):
            if kk >= 2:
                writes[kk - 2].wait()
            gathers.append(istart(kk))
            gathers[kk - 1].wait()
            writes.append(iwrite(kk - 1))
        gathers[N_ITEM_CHUNKS - 1].wait()
        writes.append(iwrite(N_ITEM_CHUNKS - 1))
        writes[N_ITEM_CHUNKS - 2].wait()
        writes[N_ITEM_CHUNKS - 1].wait()

    return k(itab, iidx_flat)


def _sc_ctx_pool(ctab, cidx_flat):
    @functools.partial(
        pl.kernel,
        mesh=_MESH,
        out_type=jax.ShapeDtypeStruct((B, 2 * E), jnp.float32),
        scratch_types=[
            pltpu.VMEM((CTX_CHUNK,), jnp.int32),
            pltpu.VMEM((CTX_CHUNK,), jnp.int32),
            pltpu.VMEM((CTX_CHUNK, E), jnp.float32),
            pltpu.VMEM((CTX_CHUNK, E), jnp.float32),
            pltpu.VMEM((B_GROUP, E), jnp.float32),
            pltpu.VMEM((B_GROUP, E), jnp.float32),
            pltpu.SemaphoreType.DMA,
            pltpu.SemaphoreType.DMA,
            pltpu.SemaphoreType.DMA,
            pltpu.SemaphoreType.DMA,
        ],
        compiler_params=_SC_PARAMS,
    )
    def k(ctab_hbm, cidx_hbm, pooled_out,
          cidx0, cidx1, cbuf0, cbuf1, pool0, pool1,
          sem_g0, sem_g1, sem_p0, sem_p1):
        wid = lax.axis_index("s") * 2 + lax.axis_index("c")
        cidx = (cidx0, cidx1)
        cbuf = (cbuf0, cbuf1)
        pool = (pool0, pool1)
        sem_g = (sem_g0, sem_g1)
        sem_p = (sem_p0, sem_p1)

        def cstart(g):
            bsel = g % 2
            cbase = pl.multiple_of(
                wid * (B_PER_TILE * S_PAD) + g * CTX_CHUNK, CTX_CHUNK)
            pltpu.sync_copy(cidx_hbm.at[pl.ds(cbase, CTX_CHUNK)], cidx[bsel])
            return pltpu.async_copy(ctab_hbm.at[cidx[bsel]], cbuf[bsel],
                                    sem_g[bsel])

        cgathers = [cstart(0)]
        pwrites = []
        for g in range(N_B_GROUPS):
            psel = g % 2
            bsel = g % 2
            cgathers[g].wait()
            if g + 1 < N_B_GROUPS:
                cgathers.append(cstart(g + 1))
            if g >= 2:
                pwrites[g - 2].wait()
            zero = jnp.zeros((16,), jnp.float32)
            for bi in range(B_GROUP):
                def body(s, acc, _bi=bi, _bsel=bsel):
                    j = _bi * S_PAD + s
                    return tuple(
                        acc[c] + cbuf[_bsel][j, pl.ds(c * 16, 16)]
                        for c in range(4))

                acc = lax.fori_loop(0, S, body, (zero, zero, zero, zero))
                for c in range(4):
                    pool[psel][bi, pl.ds(c * 16, 16)] = acc[c]
            obase = pl.multiple_of(wid * B_PER_TILE + g * B_GROUP, B_GROUP)
            pwrites.append(pltpu.async_copy(
                pool[psel],
                pooled_out.at[pl.ds(obase, B_GROUP), pl.ds(0, E)],
                sem_p[psel]))
        pwrites[N_B_GROUPS - 2].wait()
        pwrites[N_B_GROUPS - 1].wait()

    return k(ctab, cidx_flat)


def _tc_score(item2, cidx, pooled, row0, W1, b1, W2, b2):
    BB = 256

    def body(item_ref, cidx_ref, pool_ref, row0_ref, W1_ref, b1_ref,
             W2_ref, b2_ref, out_ref):
        nz = jnp.sum((cidx_ref[...] == 0).astype(jnp.float32), axis=1,
                     keepdims=True)
        pooled_c = pool_ref[...][:, :E] - nz * row0_ref[...]
        up = jnp.tanh(
            jnp.dot(pooled_c, W1_ref[...],
                    preferred_element_type=jnp.float32) + b1_ref[...])
        ctx = (jnp.dot(up, W2_ref[...], preferred_element_type=jnp.float32)
               + b2_ref[...])
        item3 = item_ref[...][:, :E].reshape(BB, S, E)
        diff = item3 - ctx[:, None, :]
        d2 = jnp.sum(diff * diff, axis=-1)
        out_ref[...] = 1.0 - jnp.tanh(jnp.sqrt(d2))

    return pl.pallas_call(
        body,
        grid=(B // BB,),
        in_specs=[
            pl.BlockSpec((BB * S, 2 * E), lambda i: (i, 0)),
            pl.BlockSpec((BB, S), lambda i: (i, 0)),
            pl.BlockSpec((BB, 2 * E), lambda i: (i, 0)),
            pl.BlockSpec((1, E), lambda i: (0, 0)),
            pl.BlockSpec((E, 2 * E), lambda i: (0, 0)),
            pl.BlockSpec((1, 2 * E), lambda i: (0, 0)),
            pl.BlockSpec((2 * E, E), lambda i: (0, 0)),
            pl.BlockSpec((1, E), lambda i: (0, 0)),
        ],
        out_specs=pl.BlockSpec((BB, S), lambda i: (i, 0)),
        out_shape=jax.ShapeDtypeStruct((B, S), jnp.float32),
    )(item2, cidx, pooled, row0, W1, b1, W2, b2)


def kernel(item_indices, context_indices, item_table, ctx_table, W1, b1, W2, b2):
    ii = item_indices.astype(jnp.int32)
    ci = context_indices.astype(jnp.int32)
    spread = (jnp.arange(B * (S_PAD - S), dtype=jnp.int32) * 7919) % NUM_ITEMS
    cidx_pad = jnp.concatenate([ci, spread.reshape(B, S_PAD - S)], axis=1)
    pooled = _sc_ctx_pool(ctx_table, cidx_pad.reshape(-1))
    item_embeds = _sc_item_gather(item_table, ii.reshape(-1))
    row0 = lax.slice(ctx_table, (0, 0), (1, E))
    return _tc_score(item_embeds, ci, pooled, row0,
                     W1, b1.reshape(1, -1), W2, b2.reshape(1, -1))
